# baseline (speedup 1.0000x reference)
import functools

import numpy as np
import jax
import jax.numpy as jnp
from jax.experimental import pallas as pl
from jax.experimental.pallas import tpu as pltpu

_N_COLORS = 16
_IMG = 224


@functools.lru_cache(maxsize=None)
def _resize_weight_mat(in_size: int, out_size: int) -> np.ndarray:
    scale = out_size / in_size
    inv_scale = 1.0 / scale
    kernel_scale = max(inv_scale, 1.0)
    sample_f = (np.arange(out_size, dtype=np.float64) + 0.5) * inv_scale - 0.5
    x = np.abs(sample_f[None, :]
               - np.arange(in_size, dtype=np.float64)[:, None]) / kernel_scale
    w = np.maximum(0.0, 1.0 - x)
    total = w.sum(axis=0, keepdims=True)
    w = np.where(np.abs(total) > 1000.0 * np.finfo(np.float32).eps,
                 w / np.where(total != 0.0, total, 1.0), 0.0)
    inside = (sample_f >= -0.5) & (sample_f <= in_size - 0.5)
    w = np.where(inside[None, :], w, 0.0)
    return np.ascontiguousarray(w, dtype=np.float32)


_BS = 16
_NH = 2


def _spec2img_body(x_ref, colors_ref, wh_ref, ww_ref, out_ref, q_ref):
    inv_n = jnp.float32(1.0 / (_N_COLORS - 1))
    offs = [colors_ref[0, c] for c in range(3)]
    steps = [(colors_ref[_N_COLORS - 1, c] - offs[c]) * inv_n
             for c in range(3)]
    j = pl.program_id(1)

    @pl.when(j == 0)
    def _quantize():
        for s in range(_BS):
            x = x_ref[s]
            mn = jnp.min(x)
            xm = x - mn
            mx = jnp.max(xm)
            q_ref[s] = jnp.round(xm * (jnp.float32(_N_COLORS - 1) / mx))

    for s in range(_BS):
        u = jnp.dot(wh_ref[...], q_ref[s], preferred_element_type=jnp.float32)
        r = jnp.dot(u, ww_ref[...], preferred_element_type=jnp.float32)
        for c in range(3):
            out_ref[s, c] = offs[c] + r * steps[c]


def kernel(inputs, colors):
    b, h, w = inputs.shape
    hh = _IMG // _NH
    wh = jnp.asarray(_resize_weight_mat(h, _IMG).T)
    ww = jnp.asarray(_resize_weight_mat(w, _IMG))
    out = pl.pallas_call(
        _spec2img_body,
        grid=(b // _BS, _NH),
        in_specs=[
            pl.BlockSpec((_BS, h, w), lambda i, j: (i, 0, 0)),
            pl.BlockSpec((_N_COLORS, 3), lambda i, j: (0, 0)),
            pl.BlockSpec((hh, h), lambda i, j: (j, 0)),
            pl.BlockSpec((w, _IMG), lambda i, j: (0, 0)),
        ],
        out_specs=pl.BlockSpec((_BS, 3, hh, _IMG), lambda i, j: (i, 0, j, 0)),
        out_shape=jax.ShapeDtypeStruct((b, 3, _IMG, _IMG), jnp.float32),
        scratch_shapes=[pltpu.VMEM((_BS, h, w), jnp.float32)],
    )(inputs, colors, wh, ww)
    return out.transpose(0, 2, 3, 1)


# speedup vs baseline: 1.9448x; 1.9448x over previous
import functools

import numpy as np
import jax
import jax.numpy as jnp
from jax.experimental import pallas as pl
from jax.experimental.pallas import tpu as pltpu

_N_COLORS = 16
_IMG = 224


@functools.lru_cache(maxsize=None)
def _resize_weight_mat(in_size: int, out_size: int) -> np.ndarray:
    scale = out_size / in_size
    inv_scale = 1.0 / scale
    kernel_scale = max(inv_scale, 1.0)
    sample_f = (np.arange(out_size, dtype=np.float64) + 0.5) * inv_scale - 0.5
    x = np.abs(sample_f[None, :]
               - np.arange(in_size, dtype=np.float64)[:, None]) / kernel_scale
    w = np.maximum(0.0, 1.0 - x)
    total = w.sum(axis=0, keepdims=True)
    w = np.where(np.abs(total) > 1000.0 * np.finfo(np.float32).eps,
                 w / np.where(total != 0.0, total, 1.0), 0.0)
    inside = (sample_f >= -0.5) & (sample_f <= in_size - 0.5)
    w = np.where(inside[None, :], w, 0.0)
    return np.ascontiguousarray(w, dtype=np.float32)


_BS = 16


def _spec2img_body(x_ref, colors_ref, wh_ref, ww_ref, out_ref):
    inv_n = jnp.float32(1.0 / (_N_COLORS - 1))
    offs = [colors_ref[0, c] for c in range(3)]
    steps = [(colors_ref[_N_COLORS - 1, c] - offs[c]) * inv_n
             for c in range(3)]
    for s in range(_BS):
        x = x_ref[s]
        mn = jnp.min(x)
        xm = x - mn
        mx = jnp.max(xm)
        q = jnp.round(xm * (jnp.float32(_N_COLORS - 1) / mx))
        u = jnp.dot(wh_ref[...], q, preferred_element_type=jnp.float32)
        r = jnp.dot(u, ww_ref[...], preferred_element_type=jnp.float32)
        for c in range(3):
            out_ref[s, c] = offs[c] + r * steps[c]


def kernel(inputs, colors):
    b, h, w = inputs.shape
    wh = jnp.asarray(_resize_weight_mat(h, _IMG).T)
    ww = jnp.asarray(_resize_weight_mat(w, _IMG))
    out = pl.pallas_call(
        _spec2img_body,
        grid=(b // _BS,),
        in_specs=[
            pl.BlockSpec((_BS, h, w), lambda i: (i, 0, 0)),
            pl.BlockSpec((_N_COLORS, 3), lambda i: (0, 0)),
            pl.BlockSpec((_IMG, h), lambda i: (0, 0)),
            pl.BlockSpec((w, _IMG), lambda i: (0, 0)),
        ],
        out_specs=pl.BlockSpec((_BS, 3, _IMG, _IMG), lambda i: (i, 0, 0, 0)),
        out_shape=jax.ShapeDtypeStruct((b, 3, _IMG, _IMG), jnp.float32),
        compiler_params=pltpu.CompilerParams(
            dimension_semantics=("parallel",)),
    )(inputs, colors, wh, ww)
    return out.transpose(0, 2, 3, 1)
